# baseline (speedup 1.0000x reference)
import numpy as np
import jax
import jax.numpy as jnp
from jax.experimental import pallas as pl

_HASH_MOD = 2**31 - 1
_N = 16384
_APAD = 128
_CHUNK = 512
_NJ = _N // _CHUNK


def _hash_body(x_ref, w_ref, o_ref):
    vals = (x_ref[...] + 1) * w_ref[...]
    s = jnp.sum(vals, axis=1, keepdims=True)
    o_ref[...] = jnp.mod(s, _HASH_MOD)


def _count_body(col_ref, row_ref, o_ref):
    j = pl.program_id(1)

    @pl.when(j == 0)
    def _():
        o_ref[0] = jnp.zeros_like(o_ref[0])

    col = col_ref[0]
    row = row_ref[...]
    eq = row == col
    o_ref[0] += jnp.sum(jnp.where(eq, 1.0, 0.0), axis=1, keepdims=True)

    @pl.when(j == _NJ - 1)
    def _():
        o_ref[0] = 1.0 / o_ref[0]


def _rank_body(col_ref, row_ref, w_ref, o_ref):
    j = pl.program_id(1)

    @pl.when(j == 0)
    def _():
        o_ref[0] = jnp.zeros_like(o_ref[0])

    col = col_ref[0]
    row = row_ref[...]
    w = w_ref[...]
    lt = row < col
    o_ref[0] += jnp.sum(jnp.where(lt, w, 0.0), axis=1, keepdims=True)


def kernel(stacked_raw_attributes, blocks_mask):
    x = stacked_raw_attributes
    n_attrs = x.shape[1]

    rng = np.random.default_rng(1234)
    weights = jnp.asarray(
        rng.integers(1, _HASH_MOD, size=(n_attrs,), dtype=np.int64).astype(np.int32) | 1
    )
    k_idx = jax.random.split(jax.random.key(42), 4)[0]
    random_index = jax.random.randint(k_idx, (), 0, blocks_mask.shape[0])
    chosen = blocks_mask[random_index]
    mw = jnp.where(chosen, weights, 0).astype(jnp.int32)
    mw_pad = jnp.zeros((1, _APAD), jnp.int32).at[0, :n_attrs].set(mw)
    x_pad = jnp.pad(x, ((0, 0), (0, _APAD - n_attrs)))

    h = pl.pallas_call(
        _hash_body,
        grid=(16,),
        in_specs=[
            pl.BlockSpec((_N // 16, _APAD), lambda i: (i, 0)),
            pl.BlockSpec((1, _APAD), lambda i: (0, 0)),
        ],
        out_specs=pl.BlockSpec((_N // 16, 1), lambda i: (i, 0)),
        out_shape=jax.ShapeDtypeStruct((_N, 1), jnp.int32),
    )(x_pad, mw_pad)

    hrow = h.reshape(1, _N)
    hcol3 = h.reshape(128, 128, 1)

    invc = pl.pallas_call(
        _count_body,
        grid=(128, _NJ),
        in_specs=[
            pl.BlockSpec((1, 128, 1), lambda i, j: (i, 0, 0)),
            pl.BlockSpec((1, _CHUNK), lambda i, j: (0, j)),
        ],
        out_specs=pl.BlockSpec((1, 128, 1), lambda i, j: (i, 0, 0)),
        out_shape=jax.ShapeDtypeStruct((128, 128, 1), jnp.float32),
    )(hcol3, hrow)
    wrow = invc.reshape(1, _N)

    ranks = pl.pallas_call(
        _rank_body,
        grid=(128, _NJ),
        in_specs=[
            pl.BlockSpec((1, 128, 1), lambda i, j: (i, 0, 0)),
            pl.BlockSpec((1, _CHUNK), lambda i, j: (0, j)),
            pl.BlockSpec((1, _CHUNK), lambda i, j: (0, j)),
        ],
        out_specs=pl.BlockSpec((1, 128, 1), lambda i, j: (i, 0, 0)),
        out_shape=jax.ShapeDtypeStruct((128, 128, 1), jnp.float32),
    )(hcol3, hrow, wrow)

    return (ranks.reshape(_N) + 0.5).astype(jnp.int32)


# speedup vs baseline: 52.5038x; 52.5038x over previous
import numpy as np
import jax
import jax.numpy as jnp
from jax.experimental import pallas as pl
from jax.experimental.pallas import tpu as pltpu

_HASH_MOD = 2**31 - 1
_APAD = 128


def _pymod(s):
    m = jax.lax.rem(s, jnp.int32(_HASH_MOD))
    return jnp.where(m < 0, m + _HASH_MOD, m)


def _roll(a, shift, axis):
    return jnp.roll(a, shift, axis=axis)


def _partner(a, j, bit0):
    d = 1 << j
    if j < 7:
        plus, minus = _roll(a, -d, 0), _roll(a, d, 0)
    else:
        dd = d >> 7
        plus, minus = _roll(a, -dd, 1), _roll(a, dd, 1)
    return jnp.where(bit0, plus, minus)


def _body(x_ref, w_ref, o_ref, h_scr):
    n = x_ref.shape[0]
    lanes = n // 128
    log_n = n.bit_length() - 1
    w = w_ref[...]

    for r in range(lanes):
        xb = x_ref[pl.ds(r * 128, 128), :]
        s = jnp.sum((xb + 1) * w, axis=1, keepdims=True)
        h_scr[:, r:r + 1] = _pymod(s)

    row = jax.lax.broadcasted_iota(jnp.int32, (128, lanes), 0)
    lane = jax.lax.broadcasted_iota(jnp.int32, (128, lanes), 1)
    flat = lane * 128 + row

    v = h_scr[...]
    u = flat
    for k in range(1, log_n + 1):
        for j in range(k - 1, -1, -1):
            bit0 = ((flat >> j) & 1) == 0
            take_min = (((flat >> k) ^ (flat >> j)) & 1) == 0
            pv = _partner(v, j, bit0)
            pu = _partner(u, j, bit0)
            take = (take_min & (pv < v)) | (~take_min & (pv > v))
            v = jnp.where(take, pv, v)
            u = jnp.where(take, pu, u)

    vr = _roll(v, 1, 0)
    vprev = jnp.where(row == 0, _roll(vr, 1, 1), vr)
    flags = jnp.where((v != vprev) & (flat != 0), 1, 0).astype(jnp.int32)
    p = flags
    for s_ in (1, 2, 4, 8, 16, 32, 64):
        p = p + jnp.where(row >= s_, _roll(p, s_, 0), 0)
    colsum = p[127:128, :]
    li = jax.lax.broadcasted_iota(jnp.int32, (1, lanes), 1)
    c = colsum
    s_ = 1
    while s_ < lanes:
        c = c + jnp.where(li >= s_, _roll(c, s_, 1), 0)
        s_ *= 2
    cexcl = jnp.where(li >= 1, _roll(c, 1, 1), 0)
    rank = p + cexcl

    q = u * n + rank
    for k in range(1, log_n + 1):
        for j in range(k - 1, -1, -1):
            bit0 = ((flat >> j) & 1) == 0
            take_min = (((flat >> k) ^ (flat >> j)) & 1) == 0
            pq = _partner(q, j, bit0)
            take = (take_min & (pq < q)) | (~take_min & (pq > q))
            q = jnp.where(take, pq, q)

    o_ref[...] = q & (n - 1)


def kernel(stacked_raw_attributes, blocks_mask):
    x = stacked_raw_attributes
    n, n_attrs = x.shape
    lanes = n // 128

    rng = np.random.default_rng(1234)
    weights = jnp.asarray(
        rng.integers(1, _HASH_MOD, size=(n_attrs,), dtype=np.int64).astype(np.int32) | 1
    )
    k_idx = jax.random.split(jax.random.key(42), 4)[0]
    random_index = jax.random.randint(k_idx, (), 0, blocks_mask.shape[0])
    chosen = blocks_mask[random_index]
    mw = jnp.where(chosen, weights, 0).astype(jnp.int32)
    mw_pad = jnp.zeros((1, _APAD), jnp.int32).at[0, :n_attrs].set(mw)
    x_pad = jnp.pad(x, ((0, 0), (0, _APAD - n_attrs)))

    out = pl.pallas_call(
        _body,
        out_shape=jax.ShapeDtypeStruct((128, lanes), jnp.int32),
        scratch_shapes=[pltpu.VMEM((128, lanes), jnp.int32)],
    )(x_pad, mw_pad)

    return out.T.reshape(n)


# speedup vs baseline: 54.8569x; 1.0448x over previous
import numpy as np
import jax
import jax.numpy as jnp
from jax.experimental import pallas as pl
from jax.experimental.pallas import tpu as pltpu

_HASH_MOD = 2**31 - 1
_APAD = 128


def _pymod(s):
    m = jax.lax.rem(s, jnp.int32(_HASH_MOD))
    return jnp.where(m < 0, m + _HASH_MOD, m)


def _roll(a, shift, axis):
    return jnp.roll(a, shift, axis=axis)


def _partner(a, j, bit0):
    d = 1 << j
    if j < 7:
        plus, minus = _roll(a, -d, 0), _roll(a, d, 0)
    else:
        dd = d >> 7
        plus, minus = _roll(a, -dd, 1), _roll(a, dd, 1)
    return jnp.where(bit0, plus, minus)


def _body(x_ref, w_ref, o_ref, h_scr):
    n = x_ref.shape[0]
    lanes = n // 128
    log_n = n.bit_length() - 1
    w = w_ref[...]

    for r in range(lanes):
        xb = x_ref[pl.ds(r * 128, 128), :]
        s = jnp.sum((xb + 1) * w, axis=1, keepdims=True)
        h_scr[:, r:r + 1] = _pymod(s)

    row = jax.lax.broadcasted_iota(jnp.int32, (128, lanes), 0)
    lane = jax.lax.broadcasted_iota(jnp.int32, (128, lanes), 1)
    flat = lane * 128 + row

    fs = [flat >> j for j in range(log_n + 1)]
    bit0s = [(fs[j] & 1) == 0 for j in range(log_n + 1)]
    v = h_scr[...]
    u = flat
    for k in range(1, log_n + 1):
        for j in range(k - 1, -1, -1):
            bit0 = bit0s[j]
            take_min = ((fs[k] ^ fs[j]) & 1) == 0
            pv = _partner(v, j, bit0)
            pu = _partner(u, j, bit0)
            vnew = jnp.where(take_min, jnp.minimum(v, pv), jnp.maximum(v, pv))
            u = jnp.where(vnew != v, pu, u)
            v = vnew

    vr = _roll(v, 1, 0)
    vprev = jnp.where(row == 0, _roll(vr, 1, 1), vr)
    flags = jnp.where((v != vprev) & (flat != 0), 1, 0).astype(jnp.int32)
    p = flags
    for s_ in (1, 2, 4, 8, 16, 32, 64):
        p = p + jnp.where(row >= s_, _roll(p, s_, 0), 0)
    colsum = p[127:128, :]
    li = jax.lax.broadcasted_iota(jnp.int32, (1, lanes), 1)
    c = colsum
    s_ = 1
    while s_ < lanes:
        c = c + jnp.where(li >= s_, _roll(c, s_, 1), 0)
        s_ *= 2
    cexcl = jnp.where(li >= 1, _roll(c, 1, 1), 0)
    rank = p + cexcl

    q = u * n + rank
    for k in range(1, log_n + 1):
        for j in range(k - 1, -1, -1):
            bit0 = bit0s[j]
            take_min = ((fs[k] ^ fs[j]) & 1) == 0
            pq = _partner(q, j, bit0)
            q = jnp.where(take_min, jnp.minimum(q, pq), jnp.maximum(q, pq))

    o_ref[...] = q & (n - 1)


def kernel(stacked_raw_attributes, blocks_mask):
    x = stacked_raw_attributes
    n, n_attrs = x.shape
    lanes = n // 128

    rng = np.random.default_rng(1234)
    weights = jnp.asarray(
        rng.integers(1, _HASH_MOD, size=(n_attrs,), dtype=np.int64).astype(np.int32) | 1
    )
    k_idx = jax.random.split(jax.random.key(42), 4)[0]
    random_index = jax.random.randint(k_idx, (), 0, blocks_mask.shape[0])
    chosen = blocks_mask[random_index]
    mw = jnp.where(chosen, weights, 0).astype(jnp.int32)
    mw_pad = jnp.zeros((1, _APAD), jnp.int32).at[0, :n_attrs].set(mw)
    x_pad = jnp.pad(x, ((0, 0), (0, _APAD - n_attrs)))

    out = pl.pallas_call(
        _body,
        out_shape=jax.ShapeDtypeStruct((128, lanes), jnp.int32),
        scratch_shapes=[pltpu.VMEM((128, lanes), jnp.int32)],
    )(x_pad, mw_pad)

    return out.T.reshape(n)


# speedup vs baseline: 69.1836x; 1.2612x over previous
import numpy as np
import jax
import jax.numpy as jnp
from jax.experimental import pallas as pl
from jax.experimental.pallas import tpu as pltpu

_HASH_MOD = 2**31 - 1
_APAD = 128


def _pymod(s):
    m = jax.lax.rem(s, jnp.int32(_HASH_MOD))
    return jnp.where(m < 0, m + _HASH_MOD, m)


def _roll(a, shift, axis):
    return jnp.roll(a, shift, axis=axis)


def _partner(a, j, bit0):
    d = 1 << j
    if j < 7:
        plus, minus = _roll(a, -d, 0), _roll(a, d, 0)
    else:
        dd = d >> 7
        plus, minus = _roll(a, -dd, 1), _roll(a, dd, 1)
    return jnp.where(bit0, plus, minus)


def _body(x_ref, w_ref, o_ref, h_scr):
    n = x_ref.shape[0]
    lanes = n // 128
    log_n = n.bit_length() - 1
    w = w_ref[...]

    for r in range(lanes):
        xb = x_ref[pl.ds(r * 128, 128), :]
        s = jnp.sum((xb + 1) * w, axis=1, keepdims=True)
        h_scr[:, r:r + 1] = _pymod(s)

    row = jax.lax.broadcasted_iota(jnp.int32, (128, lanes), 0)
    lane = jax.lax.broadcasted_iota(jnp.int32, (128, lanes), 1)
    flat = lane * 128 + row

    fs = [flat >> j for j in range(log_n + 1)]
    bit0s = [(fs[j] & 1) == 0 for j in range(log_n + 1)]
    v = h_scr[...]
    u = flat
    for k in range(1, log_n + 1):
        for j in range(k - 1, -1, -1):
            bit0 = bit0s[j]
            take_min = ((fs[k] ^ fs[j]) & 1) == 0
            pv = _partner(v, j, bit0)
            pu = _partner(u, j, bit0)
            vnew = jnp.where(take_min, jnp.minimum(v, pv), jnp.maximum(v, pv))
            u = jnp.where(vnew != v, pu, u)
            v = vnew

    vr = _roll(v, 1, 0)
    vprev = jnp.where(row == 0, _roll(vr, 1, 1), vr)
    flags = jnp.where((v != vprev) & (flat != 0), 1, 0).astype(jnp.int32)
    p = flags
    for s_ in (1, 2, 4, 8, 16, 32, 64):
        p = p + jnp.where(row >= s_, _roll(p, s_, 0), 0)
    colsum = p[127:128, :]
    li = jax.lax.broadcasted_iota(jnp.int32, (1, lanes), 1)
    c = colsum
    s_ = 1
    while s_ < lanes:
        c = c + jnp.where(li >= s_, _roll(c, s_, 1), 0)
        s_ *= 2
    cexcl = jnp.where(li >= 1, _roll(c, 1, 1), 0)
    rank = p + cexcl

    q = u * n + rank
    for k in range(1, log_n + 1):
        for j in range(k - 1, -1, -1):
            bit0 = bit0s[j]
            take_min = ((fs[k] ^ fs[j]) & 1) == 0
            pq = _partner(q, j, bit0)
            q = jnp.where(take_min, jnp.minimum(q, pq), jnp.maximum(q, pq))

    o_ref[...] = q & (n - 1)


def kernel(stacked_raw_attributes, blocks_mask):
    x = stacked_raw_attributes
    n, n_attrs = x.shape
    lanes = n // 128

    rng = np.random.default_rng(1234)
    weights = jnp.asarray(
        rng.integers(1, _HASH_MOD, size=(n_attrs,), dtype=np.int64).astype(np.int32) | 1
    )
    k_idx = jax.random.split(jax.random.key(42), 4)[0]
    random_index = jax.random.randint(k_idx, (), 0, blocks_mask.shape[0])
    chosen = blocks_mask[random_index]
    mw = jnp.where(chosen, weights, 0).astype(jnp.int32).reshape(1, n_attrs)

    out = pl.pallas_call(
        _body,
        out_shape=jax.ShapeDtypeStruct((128, lanes), jnp.int32),
        scratch_shapes=[pltpu.VMEM((128, lanes), jnp.int32)],
    )(x, mw)

    return out.T.reshape(n)
